# baseline (speedup 1.0000x reference)
import functools

import jax
import jax.numpy as jnp
from jax import lax
from jax.experimental import pallas as pl
from jax.experimental.pallas import tpu as pltpu
from jax.experimental.pallas import tpu_sc as plsc

B = 64
TOPK = 16

NC = 2
NS = 16
NW = NC * NS


def _select_body(tgt_ref, emb_ref, idx_ref, es_ref, d_ref, et_ref):
    M, C = emb_ref.shape[1], emb_ref.shape[2]
    emb = emb_ref[...]
    ns = jnp.sqrt(jnp.sum(emb * emb, axis=-1))
    emb_b = emb.astype(jnp.bfloat16)
    base = pl.program_id(0) * B
    for b in range(B):
        t_b = tgt_ref[base + b]
        er = emb_ref[b, t_b, :]
        et_ref[pl.ds(b, 1), :] = er.reshape(1, C)
        d_ref[pl.ds(b, 1), :] = lax.dot_general(
            er.reshape(1, C).astype(jnp.bfloat16), emb_b[b],
            (((1,), (1,)), ((), ())),
            preferred_element_type=jnp.float32)
    et = et_ref[...]
    n_t = jnp.sqrt(jnp.sum(et * et, axis=-1))
    s = d_ref[...] / (n_t[:, None] * ns)
    j_iota = lax.broadcasted_iota(jnp.int32, (B, M), 1)
    cols = []
    for _ in range(TOPK):
        m = jnp.max(s, axis=-1, keepdims=True)
        jk = jnp.min(jnp.where(s == m, j_iota, M), axis=-1)
        cols.append(jk.reshape(B, 1))
        s = jnp.where(j_iota == jk[:, None], -jnp.inf, s)
    idx_local = jnp.concatenate(cols, axis=1)
    row = base + lax.broadcasted_iota(jnp.int32, (B, TOPK), 0)
    idx_ref[0] = row * M + idx_local
    oh = (idx_local[:, :, None]
          == lax.broadcasted_iota(jnp.int32, (B, TOPK, M), 2)
          ).astype(jnp.bfloat16)
    for b in range(B):
        es_ref[0, b] = lax.dot_general(
            oh[b], emb_b[b], (((1,), (0,)), ((), ())),
            preferred_element_type=jnp.float32)


def _select(emb, tgt):
    N, M, C = emb.shape
    return pl.pallas_call(
        _select_body,
        grid=(N // B,),
        in_specs=[
            pl.BlockSpec(memory_space=pltpu.SMEM),
            pl.BlockSpec((B, M, C), lambda i: (i, 0, 0)),
        ],
        out_specs=[
            pl.BlockSpec((1, B, TOPK), lambda i: (i, 0, 0)),
            pl.BlockSpec((1, B, TOPK, C), lambda i: (i, 0, 0, 0)),
        ],
        out_shape=[
            jax.ShapeDtypeStruct((N // B, B, TOPK), jnp.int32),
            jax.ShapeDtypeStruct((N // B, B, TOPK, C), jnp.float32),
        ],
        scratch_shapes=[
            pltpu.VMEM((B, M), jnp.float32),
            pltpu.VMEM((B, C), jnp.float32),
        ],
    )(tgt, emb)


def _gather_x(x_flat, idx2d):
    rows_total, lanes = idx2d.shape
    rpw = rows_total // NW
    bpw = rpw * lanes
    C = x_flat.shape[1]
    out_sds = jax.ShapeDtypeStruct((rows_total * lanes, C), jnp.float32)
    mesh = plsc.VectorSubcoreMesh(core_axis_name="c", subcore_axis_name="s")

    @functools.partial(
        pl.kernel, mesh=mesh,
        compiler_params=pltpu.CompilerParams(use_tc_tiling_on_sc=False),
        out_type=out_sds,
        scratch_types=[
            pltpu.VMEM((rpw, lanes), jnp.int32),
            pltpu.VMEM((bpw, C), jnp.float32),
            pltpu.SemaphoreType.DMA,
        ],
    )
    def k(xt_hbm, idx_hbm, xo_hbm, idx_v, ox_v, sem):
        wid = lax.axis_index("s") * NC + lax.axis_index("c")
        base = wid * rpw
        pltpu.sync_copy(idx_hbm.at[pl.ds(base, rpw)], idx_v)
        hs = []
        for r in range(rpw):
            hs.append(pltpu.async_copy(
                xt_hbm.at[idx_v.at[r]],
                ox_v.at[pl.ds(r * lanes, lanes)], sem))
        for h in hs:
            h.wait()
        pltpu.sync_copy(ox_v, xo_hbm.at[pl.ds(wid * bpw, bpw)])

    return k(x_flat, idx2d)


def kernel(x, node_embedding, target_node):
    N, M, C = x.shape
    emb = lax.stop_gradient(node_embedding)
    t = target_node.astype(jnp.int32)
    flat_idx, emb_sel = _select(emb, t)
    idx2d = flat_idx.reshape(N * TOPK // 128, 128)
    x_sel = _gather_x(x.reshape(N * M, C), idx2d)
    return (x_sel.reshape(N, TOPK, C), emb_sel.reshape(N, TOPK, C))


# speedup vs baseline: 1.0873x; 1.0873x over previous
import functools

import jax
import jax.numpy as jnp
from jax import lax
from jax.experimental import pallas as pl
from jax.experimental.pallas import tpu as pltpu
from jax.experimental.pallas import tpu_sc as plsc

B = 128
TOPK = 16

NC = 2
NS = 16
NW = NC * NS


def _select_body(tgt_ref, emb_ref, idx_ref, es_ref, d_ref, et_ref):
    M, C = emb_ref.shape[1], emb_ref.shape[2]
    emb = emb_ref[...]
    ns = jnp.sqrt(jnp.sum(emb * emb, axis=-1))
    emb_b = emb.astype(jnp.bfloat16)
    base = pl.program_id(0) * B
    for b in range(B):
        t_b = tgt_ref[base + b]
        er = emb_ref[b, t_b, :]
        et_ref[pl.ds(b, 1), :] = er.reshape(1, C)
        d_ref[pl.ds(b, 1), :] = lax.dot_general(
            er.reshape(1, C).astype(jnp.bfloat16), emb_b[b],
            (((1,), (1,)), ((), ())),
            preferred_element_type=jnp.float32)
    et = et_ref[...]
    n_t = jnp.sqrt(jnp.sum(et * et, axis=-1))
    s = d_ref[...] / (n_t[:, None] * ns)
    j_iota = lax.broadcasted_iota(jnp.int32, (B, M), 1)
    cols = []
    for _ in range(TOPK):
        m = jnp.max(s, axis=-1, keepdims=True)
        jk = jnp.min(jnp.where(s == m, j_iota, M), axis=-1)
        cols.append(jk.reshape(B, 1))
        s = jnp.where(j_iota == jk[:, None], -jnp.inf, s)
    idx_local = jnp.concatenate(cols, axis=1)
    row = base + lax.broadcasted_iota(jnp.int32, (B, TOPK), 0)
    idx_ref[0] = row * M + idx_local
    oh = (idx_local[:, :, None]
          == lax.broadcasted_iota(jnp.int32, (B, TOPK, M), 2)
          ).astype(jnp.bfloat16)
    for b in range(B):
        es_ref[0, b] = lax.dot_general(
            oh[b], emb_b[b], (((1,), (0,)), ((), ())),
            preferred_element_type=jnp.float32)


def _select(emb, tgt):
    N, M, C = emb.shape
    return pl.pallas_call(
        _select_body,
        grid=(N // B,),
        in_specs=[
            pl.BlockSpec(memory_space=pltpu.SMEM),
            pl.BlockSpec((B, M, C), lambda i: (i, 0, 0)),
        ],
        out_specs=[
            pl.BlockSpec((1, B, TOPK), lambda i: (i, 0, 0)),
            pl.BlockSpec((1, B, TOPK, C), lambda i: (i, 0, 0, 0)),
        ],
        out_shape=[
            jax.ShapeDtypeStruct((N // B, B, TOPK), jnp.int32),
            jax.ShapeDtypeStruct((N // B, B, TOPK, C), jnp.float32),
        ],
        scratch_shapes=[
            pltpu.VMEM((B, M), jnp.float32),
            pltpu.VMEM((B, C), jnp.float32),
        ],
    )(tgt, emb)


def _gather_x(x_flat, idx2d):
    rows_total, lanes = idx2d.shape
    rpw = rows_total // NW
    bpw = rpw * lanes
    C = x_flat.shape[1]
    out_sds = jax.ShapeDtypeStruct((rows_total * lanes, C), jnp.float32)
    mesh = plsc.VectorSubcoreMesh(core_axis_name="c", subcore_axis_name="s")

    @functools.partial(
        pl.kernel, mesh=mesh,
        compiler_params=pltpu.CompilerParams(use_tc_tiling_on_sc=False),
        out_type=out_sds,
        scratch_types=[
            pltpu.VMEM((rpw, lanes), jnp.int32),
            pltpu.VMEM((bpw, C), jnp.float32),
            pltpu.SemaphoreType.DMA,
        ],
    )
    def k(xt_hbm, idx_hbm, xo_hbm, idx_v, ox_v, sem):
        wid = lax.axis_index("s") * NC + lax.axis_index("c")
        base = wid * rpw
        pltpu.sync_copy(idx_hbm.at[pl.ds(base, rpw)], idx_v)
        hs = []
        for r in range(rpw):
            hs.append(pltpu.async_copy(
                xt_hbm.at[idx_v.at[r]],
                ox_v.at[pl.ds(r * lanes, lanes)], sem))
        for h in hs:
            h.wait()
        pltpu.sync_copy(ox_v, xo_hbm.at[pl.ds(wid * bpw, bpw)])

    return k(x_flat, idx2d)


def kernel(x, node_embedding, target_node):
    N, M, C = x.shape
    emb = lax.stop_gradient(node_embedding)
    t = target_node.astype(jnp.int32)
    flat_idx, emb_sel = _select(emb, t)
    idx2d = flat_idx.reshape(N * TOPK // 128, 128)
    x_sel = _gather_x(x.reshape(N * M, C), idx2d)
    return (x_sel.reshape(N, TOPK, C), emb_sel.reshape(N, TOPK, C))
